# baseline (speedup 1.0000x reference)
import functools

import jax
import jax.numpy as jnp
from jax import lax
from jax.experimental import pallas as pl
from jax.experimental.pallas import tpu as pltpu
from jax.experimental.pallas import tpu_sc as plsc

_K = 256
_M = 8
_IVF_M = 5
_IVF_K = 1024
_B = 16384
_D = 64
_NC = 2
_NS = 16
_NW = _NC * _NS
_BPW = _B // _NW
_CB = 128
_NCH = _BPW // _CB


def _sc_decode(codes_MB, ivf_codes, codebook_MKD, ivf_map_flat, cmv_flat):
  mesh = plsc.VectorSubcoreMesh(core_axis_name="c", subcore_axis_name="s")

  @functools.partial(
      pl.kernel,
      out_type=jax.ShapeDtypeStruct((_B, _D), jnp.float32),
      mesh=mesh,
      scratch_types=[
          pltpu.VMEM(((_M + _IVF_M) * _BPW,), jnp.int32),
          pltpu.VMEM((_BPW,), jnp.int32),
          pltpu.VMEM((_IVF_K * _IVF_M,), jnp.int32),
          pltpu.VMEM((16,), jnp.int32),
          pltpu.VMEM((_M * _BPW,), jnp.int32),
          pltpu.VMEM((_M, _CB, _D), jnp.float32),
          pltpu.SemaphoreType.DMA,
      ],
      compiler_params=pltpu.CompilerParams(
          needs_layout_passes=False, use_tc_tiling_on_sc=False),
  )
  def k(codes_hbm, ivfc_hbm, book_hbm, map_hbm, cmv_hbm, out_hbm,
        call_v, ivfc_v, map_v, cmv_v, comb_v, gbuf, sem):
    wid = lax.axis_index("s") * _NC + lax.axis_index("c")
    base = wid * _BPW

    for m in range(_M):
      pltpu.sync_copy(codes_hbm.at[m, pl.ds(base, _BPW)],
                      call_v.at[pl.ds(m * _BPW, _BPW)])
    pltpu.sync_copy(ivfc_hbm.at[pl.ds(base, _BPW)], ivfc_v)
    pltpu.sync_copy(map_hbm, map_v)
    pltpu.sync_copy(cmv_hbm, cmv_v)

    cmv = cmv_v[pl.ds(0, 16)]
    c0 = [cmv[m] for m in range(_M)]
    c1 = [cmv[_M + m] for m in range(_M)]
    lane = lax.iota(jnp.int32, 16)

    def idx_body(i, carry):
      iv = ivfc_v[pl.ds(i * 16, 16)]
      for j in range(_IVF_M):
        row = plsc.load_gather(map_v, [iv * _IVF_M + j])
        call_v[pl.ds((_M + j) * _BPW + i * 16, 16)] = row
      off = i * 16 + lane
      for m in range(_M):
        r0 = plsc.load_gather(call_v, [c0[m] * _BPW + off])
        r1 = plsc.load_gather(call_v, [c1[m] * _BPW + off])
        comb_v[pl.ds(m * _BPW + i * 16, 16)] = r0 * _K + r1
      return carry

    lax.fori_loop(0, _BPW // 16, idx_body, 0)

    for t in range(_NCH):
      cps = []
      for m in range(_M):
        idx = comb_v.at[pl.ds(m * _BPW + t * _CB, _CB)]
        cps.append(pltpu.async_copy(book_hbm.at[m].at[idx], gbuf.at[m], sem))
      for cp in cps:
        cp.wait()

      def acc_body(r, carry):
        for c in range(_D // 16):
          sl = pl.ds(c * 16, 16)
          acc = gbuf[0, r, sl]
          for m in range(1, _M):
            acc = acc + gbuf[m, r, sl]
          gbuf[0, r, sl] = acc
        return carry

      lax.fori_loop(0, _CB, acc_body, 0)
      pltpu.sync_copy(gbuf.at[0], out_hbm.at[pl.ds(base + t * _CB, _CB)])

  return k(codes_MB, ivf_codes, codebook_MKD, ivf_map_flat, cmv_flat)


def kernel(codes_MB, ivf_codes, codebook_MKD, ivf_code_map, combine_mvals_m):
  return _sc_decode(codes_MB, ivf_codes, codebook_MKD,
                    ivf_code_map.reshape(-1), combine_mvals_m.reshape(-1))


# speedup vs baseline: 1.3485x; 1.3485x over previous
import functools

import jax
import jax.numpy as jnp
from jax import lax
from jax.experimental import pallas as pl
from jax.experimental.pallas import tpu as pltpu
from jax.experimental.pallas import tpu_sc as plsc

_K = 256
_M = 8
_IVF_M = 5
_IVF_K = 1024
_B = 16384
_D = 64
_KT = 65536
_NC = 2
_NS = 16
_BPS = _B // _NS


def _sc_decode(codes_MB, ivf_codes, book6, ivf_map_flat, cmv_flat):
  mesh = plsc.VectorSubcoreMesh(core_axis_name="c", subcore_axis_name="s")

  @functools.partial(
      pl.kernel,
      out_type=jax.ShapeDtypeStruct((8, 128, 8, 128), jnp.float32),
      mesh=mesh,
      scratch_types=[
          pltpu.VMEM(((_M + _IVF_M) * _BPS,), jnp.int32),
          pltpu.VMEM((_BPS,), jnp.int32),
          pltpu.VMEM((_IVF_K * _IVF_M,), jnp.int32),
          pltpu.VMEM((16,), jnp.int32),
          pltpu.VMEM((_M * _BPS,), jnp.int32),
          pltpu.VMEM((512, 128), jnp.float32),
          pltpu.VMEM((4096,), jnp.int32),
          pltpu.VMEM((128, 128), jnp.float32),
          pltpu.VMEM_SHARED((_M, _B), jnp.int32),
      ],
      compiler_params=pltpu.CompilerParams(
          needs_layout_passes=False, use_tc_tiling_on_sc=False),
  )
  def k(codes_hbm, ivfc_hbm, book_hbm, map_hbm, cmv_hbm, out_hbm,
        call_v, ivfc_v, map_v, cmv_v, comb_v, table_v, idx_v, acc_v, comb_sp):
    cid = lax.axis_index("c")
    sid = lax.axis_index("s")
    base = sid * _BPS

    for m in range(_M):
      pltpu.sync_copy(codes_hbm.at[m, pl.ds(base, _BPS)],
                      call_v.at[pl.ds(m * _BPS, _BPS)])
    pltpu.sync_copy(ivfc_hbm.at[pl.ds(base, _BPS)], ivfc_v)
    pltpu.sync_copy(map_hbm, map_v)
    pltpu.sync_copy(cmv_hbm, cmv_v)

    cmv = cmv_v[pl.ds(0, 16)]
    c0 = [cmv[m] for m in range(_M)]
    c1 = [cmv[_M + m] for m in range(_M)]
    lane = lax.iota(jnp.int32, 16)

    def idx_body(i, carry):
      iv = ivfc_v[pl.ds(i * 16, 16)]
      for j in range(_IVF_M):
        row = plsc.load_gather(map_v, [iv * _IVF_M + j])
        call_v[pl.ds((_M + j) * _BPS + i * 16, 16)] = row
      off = i * 16 + lane
      for m in range(_M):
        r0 = plsc.load_gather(call_v, [c0[m] * _BPS + off])
        r1 = plsc.load_gather(call_v, [c1[m] * _BPS + off])
        comb_v[pl.ds(m * _BPS + i * 16, 16)] = r0 * _K + r1
      return carry

    lax.fori_loop(0, _BPS // 16, idx_body, 0)

    for m in range(_M):
      pltpu.sync_copy(comb_v.at[pl.ds(m * _BPS, _BPS)],
                      comb_sp.at[m, pl.ds(base, _BPS)])
    plsc.subcore_barrier()

    for j in range(2):
      d = cid * 32 + sid * 2 + j
      dg = lax.shift_right_logical(d, 3)
      din = lax.bitwise_and(d, 7)
      for m in range(_M):
        pltpu.sync_copy(book_hbm.at[m, dg, :, din, :], table_v)
        for q in range(4):
          pltpu.sync_copy(comb_sp.at[m, pl.ds(q * 4096, 4096)], idx_v)

          def gat_body(i, carry, first=(m == 0), qbase=q * 256):
            for u in range(4):
              ii = i * 4 + u
              gi = qbase + ii
              row = gi >> 3
              col = (gi & 7) * 16
              idx16 = idx_v[pl.ds(ii * 16, 16)]
              hi = lax.shift_right_logical(idx16, 7)
              lo = lax.bitwise_and(idx16, 127)
              val = plsc.load_gather(table_v, [hi, lo])
              if first:
                acc_v[row, pl.ds(col, 16)] = val
              else:
                acc_v[row, pl.ds(col, 16)] = acc_v[row, pl.ds(col, 16)] + val
            return carry

          lax.fori_loop(0, 64, gat_body, 0)
      pltpu.sync_copy(acc_v, out_hbm.at[dg, :, din, :])

  return k(codes_MB, ivf_codes, book6, ivf_map_flat, cmv_flat)


def kernel(codes_MB, ivf_codes, codebook_MKD, ivf_code_map, combine_mvals_m):
  book6 = codebook_MKD.reshape(8, 512, 128, 8, 8).transpose(0, 3, 1, 4, 2)
  out6 = _sc_decode(codes_MB, ivf_codes, book6,
                    ivf_code_map.reshape(-1), combine_mvals_m.reshape(-1))
  return out6.transpose(1, 3, 0, 2).reshape(_B, _D)
